# Initial kernel scaffold; baseline (speedup 1.0000x reference)
import jax
import jax.numpy as jnp
from jax.experimental import pallas as pl


def kernel(syn_travel, syn_value, vm, acc, input_current, L_e, W_e, phase, src, tgt):
    raise NotImplementedError("write your pallas kernel here")


# speedup vs baseline: 3.7659x; 3.7659x over previous
import functools

import jax
import jax.numpy as jnp
from jax import lax
from jax.experimental import pallas as pl
from jax.experimental.pallas import tpu as pltpu
from jax.experimental.pallas import tpu_sc as plsc

_TAU = 10.0
_DT = 1.0
_THRESH = 0.5
_VMAX = 1.0
_ATOL = 1e-5
_RTOL = 1e-8

_EB_A = 4096
_EB_C = 3072
_SC_CHUNK = 2048
_UNROLL = 4

_NC = 2
_NS = 16
_LANES = 16
_NW = _NC * _NS


def _mesh():
    return plsc.VectorSubcoreMesh(
        core_axis_name="c", subcore_axis_name="s",
        num_cores=_NC, num_subcores=_NS)


def _sc_params():
    return pltpu.CompilerParams(needs_layout_passes=False)


def _spikes_kernel(st_ref, sv_ref, l_ref, w_ref, spk_ref):
    st = st_ref[...]
    lvals = l_ref[0]
    arrived = jnp.abs(st - lvals) <= (_ATOL + _RTOL * jnp.abs(lvals))
    spk_ref[...] = jnp.where(arrived, sv_ref[...] * w_ref[0], 0.0)


def _make_sc_scatter(b, e, n, c):
    nchunks = e // c
    assert nchunks % 2 == 0
    rows_per = b // _NW
    nvec = c // _LANES

    @functools.partial(
        pl.kernel, mesh=_mesh(),
        out_type=jax.ShapeDtypeStruct((b * n,), jnp.float32),
        scratch_types=[
            pltpu.VMEM((2, c), jnp.int32),
            pltpu.VMEM((2, rows_per, c), jnp.float32),
            pltpu.VMEM((_LANES * n,), jnp.float32),
            pltpu.VMEM((_LANES * n,), jnp.float32),
            pltpu.VMEM((n,), jnp.float32),
            pltpu.SemaphoreType.DMA,
            pltpu.SemaphoreType.DMA,
        ],
        compiler_params=_sc_params(),
    )
    def k(spk_hbm, tgt_hbm, isyn_hbm, tgt_v, spk_v, acc0, acc1, red_v,
          sem0, sem1):
        wid = lax.axis_index("s") * _NC + lax.axis_index("c")
        b0 = wid * rows_per
        lane = lax.iota(jnp.int32, _LANES)
        sems = (sem0, sem1)

        zero = jnp.zeros((_LANES,), jnp.float32)

        def zbody(i, _):
            acc0[pl.ds(i * _LANES, _LANES)] = zero
            acc1[pl.ds(i * _LANES, _LANES)] = zero
            return 0
        lax.fori_loop(0, _LANES * n // _LANES, zbody, 0)

        def start(ci, p):
            base = ci * c
            pltpu.async_copy(tgt_hbm.at[pl.ds(base, c)], tgt_v.at[p],
                             sems[p])
            pltpu.async_copy(spk_hbm.at[pl.ds(b0 * e + base, c)],
                             spk_v.at[p, 0], sems[p])
            pltpu.async_copy(spk_hbm.at[pl.ds((b0 + 1) * e + base, c)],
                             spk_v.at[p, 1], sems[p])

        def wait(p):
            pltpu.make_async_copy(tgt_hbm.at[pl.ds(0, c)], tgt_v.at[p],
                                  sems[p]).wait()
            pltpu.make_async_copy(spk_hbm.at[pl.ds(0, c)], spk_v.at[p, 0],
                                  sems[p]).wait()
            pltpu.make_async_copy(spk_hbm.at[pl.ds(0, c)], spk_v.at[p, 1],
                                  sems[p]).wait()

        def compute(p):
            def jbody(j, _):
                for u in range(_UNROLL):
                    off = (j * _UNROLL + u) * _LANES
                    idx = tgt_v[p, pl.ds(off, _LANES)]
                    addr = lane * n + idx
                    plsc.addupdate_scatter(acc0, [addr],
                                           spk_v[p, 0, pl.ds(off, _LANES)])
                    plsc.addupdate_scatter(acc1, [addr],
                                           spk_v[p, 1, pl.ds(off, _LANES)])
                return 0
            lax.fori_loop(0, nvec // _UNROLL, jbody, 0)

        start(0, 0)

        def pair_body(i, _):
            start(2 * i + 1, 1)
            wait(0)
            compute(0)

            @pl.when(2 * i + 2 < nchunks)
            def _():
                start(2 * i + 2, 0)
            wait(1)
            compute(1)
            return 0
        lax.fori_loop(0, nchunks // 2, pair_body, 0)

        for r, acc in ((0, acc0), (1, acc1)):
            def rbody(g, _, acc=acc):
                s = acc[pl.ds(g * _LANES, _LANES)]
                for l in range(1, _LANES):
                    s = s + acc[pl.ds(l * n + g * _LANES, _LANES)]
                red_v[pl.ds(g * _LANES, _LANES)] = s
                return 0
            lax.fori_loop(0, n // _LANES, rbody, 0)
            pltpu.sync_copy(red_v, isyn_hbm.at[pl.ds((b0 + r) * n, n)])

    return k


def _neuron_kernel(nh, no, tail_pad, isyn_ref, vm_ref, acc_ref, inp_ref,
                   phase_ref, vexc_ref, tail_ref):
    inject = (phase_ref[...] == 2).astype(jnp.float32)
    inp = inp_ref[...]
    b = inp.shape[0]
    i_inj = jnp.concatenate(
        [inp * inject, jnp.zeros((b, no), jnp.float32)], axis=1)
    i_syn = isyn_ref[...] + i_inj
    vm = vm_ref[...]
    vm1 = vm + (i_syn - vm) * (_DT / _TAU)
    v_exc = jnp.maximum(0.0, vm1 - _THRESH)
    fired = (v_exc > 0).astype(jnp.float32)
    vm2 = vm1 - vm1 * fired + 0.2 * fired
    acc1 = acc_ref[...] + vm1[:, -no:]
    spike_rate = jnp.mean(fired, axis=1, keepdims=True)
    input_norm = jnp.sqrt(jnp.sum(inp * inp, axis=1, keepdims=True)) * inject
    vexc_ref[...] = v_exc
    tail_ref[...] = jnp.concatenate(
        [vm2, acc1, inject, spike_rate, input_norm,
         jnp.zeros((b, tail_pad), jnp.float32)], axis=1)


def _make_sc_gather(b, e, n, c):
    nchunks = e // c
    assert nchunks % 2 == 0
    rows_per = b // _NW
    nvec = c // _LANES

    @functools.partial(
        pl.kernel, mesh=_mesh(),
        out_type=jax.ShapeDtypeStruct((b * e,), jnp.float32),
        scratch_types=[
            pltpu.VMEM((2, c), jnp.int32),
            pltpu.VMEM((n,), jnp.float32),
            pltpu.VMEM((n,), jnp.float32),
            pltpu.VMEM((2, rows_per, c), jnp.float32),
            pltpu.SemaphoreType.DMA,
            pltpu.SemaphoreType.DMA,
            pltpu.SemaphoreType.DMA,
            pltpu.SemaphoreType.DMA,
        ],
        compiler_params=_sc_params(),
    )
    def k(vexc_hbm, src_hbm, g_hbm, src_v, vex0, vex1, gbuf, sem0, sem1,
          osem0, osem1):
        wid = lax.axis_index("s") * _NC + lax.axis_index("c")
        b0 = wid * rows_per
        sems = (sem0, sem1)
        osems = (osem0, osem1)
        pltpu.sync_copy(vexc_hbm.at[pl.ds(b0 * n, n)], vex0)
        pltpu.sync_copy(vexc_hbm.at[pl.ds((b0 + 1) * n, n)], vex1)

        def start(ci, p):
            pltpu.async_copy(src_hbm.at[pl.ds(ci * c, c)], src_v.at[p],
                             sems[p])

        def wait(p):
            pltpu.make_async_copy(src_hbm.at[pl.ds(0, c)], src_v.at[p],
                                  sems[p]).wait()

        def out_start(ci, p):
            base = ci * c
            pltpu.async_copy(gbuf.at[p, 0], g_hbm.at[pl.ds(b0 * e + base, c)],
                             osems[p])
            pltpu.async_copy(gbuf.at[p, 1],
                             g_hbm.at[pl.ds((b0 + 1) * e + base, c)],
                             osems[p])

        def out_wait(p):
            pltpu.make_async_copy(gbuf.at[p, 0], g_hbm.at[pl.ds(0, c)],
                                  osems[p]).wait()
            pltpu.make_async_copy(gbuf.at[p, 1], g_hbm.at[pl.ds(0, c)],
                                  osems[p]).wait()

        def compute(p):
            def jbody(j, _):
                for u in range(_UNROLL):
                    off = (j * _UNROLL + u) * _LANES
                    idx = src_v[p, pl.ds(off, _LANES)]
                    gbuf[p, 0, pl.ds(off, _LANES)] = plsc.load_gather(
                        vex0, [idx])
                    gbuf[p, 1, pl.ds(off, _LANES)] = plsc.load_gather(
                        vex1, [idx])
                return 0
            lax.fori_loop(0, nvec // _UNROLL, jbody, 0)

        start(0, 0)

        def pair_body(i, _):
            start(2 * i + 1, 1)
            wait(0)

            @pl.when(i > 0)
            def _():
                out_wait(0)
            compute(0)
            out_start(2 * i, 0)

            @pl.when(2 * i + 2 < nchunks)
            def _():
                start(2 * i + 2, 0)
            wait(1)

            @pl.when(i > 0)
            def _():
                out_wait(1)
            compute(1)
            out_start(2 * i + 1, 1)
            return 0
        lax.fori_loop(0, nchunks // 2, pair_body, 0)
        out_wait(0)
        out_wait(1)

    return k


def _pass2_kernel(n_st_blocks, st_ref, sv_ref, l_ref, g_ref, tail_ref,
                  out_ref):
    i = pl.program_id(0)
    st = st_ref[...]
    sv = sv_ref[...]
    lvals = l_ref[0]
    g = g_ref[...]

    arrived = jnp.abs(st - lvals) <= (_ATOL + _RTOL * jnp.abs(lvals))
    stz = jnp.where(arrived, 0.0, st)
    svz = jnp.where(arrived, 0.0, sv)

    new = (g > 0) & (st == 0)
    st2 = stz + jnp.where(stz > 0, _DT * _VMAX, 0.0) \
              + jnp.where(new, _DT * _VMAX, 0.0)
    sv2 = svz + jnp.where(new, g, 0.0)

    @pl.when(i < n_st_blocks)
    def _():
        out_ref[...] = st2

    @pl.when((i >= n_st_blocks) & (i < 2 * n_st_blocks))
    def _():
        out_ref[...] = sv2

    @pl.when(i == 2 * n_st_blocks)
    def _():
        out_ref[...] = tail_ref[...]


def kernel(syn_travel, syn_value, vm, acc, input_current, L_e, W_e, phase,
           src, tgt):
    b, e = syn_travel.shape
    n = vm.shape[1]
    nh = input_current.shape[1]
    no = acc.shape[1]
    f32 = jnp.float32

    eba = _EB_A
    nblk_a = e // eba
    l3 = L_e.reshape(nblk_a, 1, eba)
    w3 = W_e.reshape(nblk_a, 1, eba)
    spikes = pl.pallas_call(
        _spikes_kernel,
        grid=(nblk_a,),
        in_specs=[
            pl.BlockSpec((b, eba), lambda i: (0, i)),
            pl.BlockSpec((b, eba), lambda i: (0, i)),
            pl.BlockSpec((1, 1, eba), lambda i: (i, 0, 0)),
            pl.BlockSpec((1, 1, eba), lambda i: (i, 0, 0)),
        ],
        out_specs=pl.BlockSpec((b, eba), lambda i: (0, i)),
        out_shape=jax.ShapeDtypeStruct((b, e), f32),
    )(syn_travel, syn_value, l3, w3)

    isyn = _make_sc_scatter(b, e, n, _SC_CHUNK)(
        spikes.reshape(-1), tgt).reshape(b, n)

    tail_cols = n + no + 3
    tail_pad = _EB_C - tail_cols
    vexc, tail = pl.pallas_call(
        functools.partial(_neuron_kernel, nh, no, tail_pad),
        out_shape=[
            jax.ShapeDtypeStruct((b, n), f32),
            jax.ShapeDtypeStruct((b, _EB_C), f32),
        ],
    )(isyn, vm, acc, input_current, phase.reshape(b, 1))

    gathered = _make_sc_gather(b, e, n, _SC_CHUNK)(
        vexc.reshape(-1), src).reshape(b, e)

    ebc = _EB_C
    nblk_c = e // ebc
    out_cols = 2 * e + tail_cols
    l3c = L_e.reshape(nblk_c, 1, ebc)

    def edge_map2(i):
        j = jnp.where(i < nblk_c, i, i - nblk_c)
        return (0, jnp.minimum(j, nblk_c - 1))

    def edge_map3(i):
        j = jnp.where(i < nblk_c, i, i - nblk_c)
        return (jnp.minimum(j, nblk_c - 1), 0, 0)

    out = pl.pallas_call(
        functools.partial(_pass2_kernel, nblk_c),
        grid=(2 * nblk_c + 1,),
        in_specs=[
            pl.BlockSpec((b, ebc), edge_map2),
            pl.BlockSpec((b, ebc), edge_map2),
            pl.BlockSpec((1, 1, ebc), edge_map3),
            pl.BlockSpec((b, ebc), edge_map2),
            pl.BlockSpec((b, ebc), lambda i: (0, 0)),
        ],
        out_specs=pl.BlockSpec((b, ebc), lambda i: (0, i)),
        out_shape=jax.ShapeDtypeStruct((b, out_cols), f32),
    )(syn_travel, syn_value, l3c, gathered, tail)

    return out


# speedup vs baseline: 5.5158x; 1.4647x over previous
import functools

import jax
import jax.numpy as jnp
from jax import lax
from jax.experimental import pallas as pl
from jax.experimental.pallas import tpu as pltpu
from jax.experimental.pallas import tpu_sc as plsc

_TAU = 10.0
_DT = 1.0
_THRESH = 0.5
_VMAX = 1.0
_ATOL = 1e-5
_RTOL = 1e-8

_EB_A = 4096
_EB_C = 3072

_NC = 2
_NS = 16
_NW = _NC * _NS
_IR = 128


def _mesh():
    return plsc.VectorSubcoreMesh(
        core_axis_name="c", subcore_axis_name="s",
        num_cores=_NC, num_subcores=_NS)


def _sc_params():
    return pltpu.CompilerParams(needs_layout_passes=False, use_tc_tiling_on_sc=False)


def _spikes_kernel(st_ref, sv_ref, l_ref, w_ref, spkt_ref):
    st = st_ref[...]
    lvals = l_ref[0]
    arrived = jnp.abs(st - lvals) <= (_ATOL + _RTOL * jnp.abs(lvals))
    spikes = jnp.where(arrived, sv_ref[...] * w_ref[0], 0.0)
    spkt_ref[...] = spikes.T


def _make_sc_scatter(b, e, n, ce):
    eper = e // _NW
    nchunks = eper // ce
    assert nchunks % 2 == 0
    kr = ce // _IR
    zrows = n // _NS
    zc = 8
    assert zrows % zc == 0

    @functools.partial(
        pl.kernel, mesh=_mesh(),
        out_type=jax.ShapeDtypeStruct((_NC, n, b), jnp.float32),
        scratch_types=[
            pltpu.VMEM((2, ce, b), jnp.float32),
            pltpu.VMEM((2, kr, _IR), jnp.int32),
            pltpu.VMEM((zc, b), jnp.float32),
            pltpu.VMEM_SHARED((n, b), jnp.float32),
            pltpu.SemaphoreType.DMA,
            pltpu.SemaphoreType.DMA,
            pltpu.SemaphoreType.DMA,
            pltpu.SemaphoreType.DMA,
        ],
        compiler_params=_sc_params(),
    )
    def k(spkt_hbm, tgt3_hbm, isynt_hbm, val_v, idx_v, zbuf, shared,
          in0, in1, sc0, sc1):
        cid = lax.axis_index("c")
        sid = lax.axis_index("s")
        wid = sid * _NC + cid
        eb0 = wid * eper
        rb0 = eb0 // _IR
        insems = (in0, in1)
        scsems = (sc0, sc1)

        for r in range(zc):
            for q in range(b // 16):
                zbuf[r, pl.ds(q * 16, 16)] = jnp.zeros((16,), jnp.float32)
        for z in range(zrows // zc):
            pltpu.sync_copy(
                zbuf, shared.at[pl.ds(sid * zrows + z * zc, zc)])
        plsc.subcore_barrier()

        def start_in(ci, p):
            pltpu.async_copy(spkt_hbm.at[pl.ds(eb0 + ci * ce, ce)],
                             val_v.at[p], insems[p])
            pltpu.async_copy(tgt3_hbm.at[pl.ds(rb0 + ci * kr, kr)],
                             idx_v.at[p], insems[p])

        def wait_in(p):
            pltpu.make_async_copy(spkt_hbm.at[pl.ds(0, ce)], val_v.at[p],
                                  insems[p]).wait()
            pltpu.make_async_copy(tgt3_hbm.at[pl.ds(0, kr)], idx_v.at[p],
                                  insems[p]).wait()

        def issue_scatter(p):
            for j in range(kr):
                pltpu.async_copy(val_v.at[p, pl.ds(j * _IR, _IR)],
                                 shared.at[idx_v.at[p, j]], scsems[p],
                                 add=True)

        def drain_scatter(p):
            for j in range(kr):
                pltpu.make_async_copy(val_v.at[p, pl.ds(j * _IR, _IR)],
                                      shared.at[idx_v.at[p, j]],
                                      scsems[p]).wait()

        start_in(0, 0)

        def pair_body(i, _):
            start_in(2 * i + 1, 1)
            wait_in(0)
            issue_scatter(0)
            drain_scatter(0)

            @pl.when(2 * i + 2 < nchunks)
            def _():
                start_in(2 * i + 2, 0)
            wait_in(1)
            issue_scatter(1)
            drain_scatter(1)
            return 0
        lax.fori_loop(0, nchunks // 2, pair_body, 0)

        plsc.subcore_barrier()

        @pl.when(sid == 0)
        def _():
            pltpu.sync_copy(shared, isynt_hbm.at[cid])

    return k


def _neuron_kernel(nh, no, tail_pad, isynt_ref, vm_ref, acc_ref, inp_ref,
                   phase_ref, vexc_ref, vexct_ref, tail_ref):
    inject = (phase_ref[...] == 2).astype(jnp.float32)
    inp = inp_ref[...]
    b = inp.shape[0]
    i_syn = (isynt_ref[0] + isynt_ref[1]).T
    i_inj = jnp.concatenate(
        [inp * inject, jnp.zeros((b, no), jnp.float32)], axis=1)
    i_syn = i_syn + i_inj
    vm = vm_ref[...]
    vm1 = vm + (i_syn - vm) * (_DT / _TAU)
    v_exc = jnp.maximum(0.0, vm1 - _THRESH)
    fired = (v_exc > 0).astype(jnp.float32)
    vm2 = vm1 - vm1 * fired + 0.2 * fired
    acc1 = acc_ref[...] + vm1[:, -no:]
    spike_rate = jnp.mean(fired, axis=1, keepdims=True)
    input_norm = jnp.sqrt(jnp.sum(inp * inp, axis=1, keepdims=True)) * inject
    vexc_ref[...] = v_exc
    vexct_ref[...] = v_exc.T
    tail_ref[...] = jnp.concatenate(
        [vm2, acc1, inject, spike_rate, input_norm,
         jnp.zeros((b, tail_pad), jnp.float32)], axis=1)


def _make_sc_gather(b, e, n, ce):
    eper = e // _NW
    nchunks = eper // ce
    assert nchunks % 2 == 0
    kr = ce // _IR
    srows = n // _NS

    @functools.partial(
        pl.kernel, mesh=_mesh(),
        out_type=jax.ShapeDtypeStruct((e, b), jnp.float32),
        scratch_types=[
            pltpu.VMEM((2, ce, b), jnp.float32),
            pltpu.VMEM((2, kr, _IR), jnp.int32),
            pltpu.VMEM_SHARED((n, b), jnp.float32),
            pltpu.SemaphoreType.DMA,
            pltpu.SemaphoreType.DMA,
            pltpu.SemaphoreType.DMA,
            pltpu.SemaphoreType.DMA,
            pltpu.SemaphoreType.DMA,
            pltpu.SemaphoreType.DMA,
        ],
        compiler_params=_sc_params(),
    )
    def k(vexct_hbm, src3_hbm, gt_hbm, g_v, idx_v, shared,
          in0, in1, ga0, ga1, out0, out1):
        cid = lax.axis_index("c")
        sid = lax.axis_index("s")
        wid = sid * _NC + cid
        eb0 = wid * eper
        rb0 = eb0 // _IR
        insems = (in0, in1)
        gasems = (ga0, ga1)
        outsems = (out0, out1)

        pltpu.sync_copy(vexct_hbm.at[pl.ds(sid * srows, srows)],
                        shared.at[pl.ds(sid * srows, srows)])
        plsc.subcore_barrier()

        def start_in(ci, p):
            pltpu.async_copy(src3_hbm.at[pl.ds(rb0 + ci * kr, kr)],
                             idx_v.at[p], insems[p])

        def wait_in(p):
            pltpu.make_async_copy(src3_hbm.at[pl.ds(0, kr)], idx_v.at[p],
                                  insems[p]).wait()

        def issue_gather(p):
            for j in range(kr):
                pltpu.async_copy(shared.at[idx_v.at[p, j]],
                                 g_v.at[p, pl.ds(j * _IR, _IR)], gasems[p])

        def drain_gather(p):
            for j in range(kr):
                pltpu.make_async_copy(shared.at[idx_v.at[p, j]],
                                      g_v.at[p, pl.ds(j * _IR, _IR)],
                                      gasems[p]).wait()

        def start_out(ci, p):
            pltpu.async_copy(g_v.at[p], gt_hbm.at[pl.ds(eb0 + ci * ce, ce)],
                             outsems[p])

        def drain_out(p):
            pltpu.make_async_copy(g_v.at[p], gt_hbm.at[pl.ds(0, ce)],
                                  outsems[p]).wait()

        start_in(0, 0)

        def pair_body(i, _):
            start_in(2 * i + 1, 1)
            wait_in(0)

            @pl.when(i > 0)
            def _():
                drain_out(0)
            issue_gather(0)
            drain_gather(0)
            start_out(2 * i, 0)

            @pl.when(2 * i + 2 < nchunks)
            def _():
                start_in(2 * i + 2, 0)
            wait_in(1)

            @pl.when(i > 0)
            def _():
                drain_out(1)
            issue_gather(1)
            drain_gather(1)
            start_out(2 * i + 1, 1)
            return 0
        lax.fori_loop(0, nchunks // 2, pair_body, 0)
        drain_out(0)
        drain_out(1)

    return k


def _pass2_kernel(n_st_blocks, st_ref, sv_ref, l_ref, gt_ref, tail_ref,
                  out_ref):
    i = pl.program_id(0)
    st = st_ref[...]
    sv = sv_ref[...]
    lvals = l_ref[0]
    g = gt_ref[...].T

    arrived = jnp.abs(st - lvals) <= (_ATOL + _RTOL * jnp.abs(lvals))
    stz = jnp.where(arrived, 0.0, st)
    svz = jnp.where(arrived, 0.0, sv)

    new = (g > 0) & (st == 0)
    st2 = stz + jnp.where(stz > 0, _DT * _VMAX, 0.0) \
              + jnp.where(new, _DT * _VMAX, 0.0)
    sv2 = svz + jnp.where(new, g, 0.0)

    @pl.when(i < n_st_blocks)
    def _():
        out_ref[...] = st2

    @pl.when((i >= n_st_blocks) & (i < 2 * n_st_blocks))
    def _():
        out_ref[...] = sv2

    @pl.when(i == 2 * n_st_blocks)
    def _():
        out_ref[...] = tail_ref[...]


def kernel(syn_travel, syn_value, vm, acc, input_current, L_e, W_e, phase,
           src, tgt):
    b, e = syn_travel.shape
    n = vm.shape[1]
    nh = input_current.shape[1]
    no = acc.shape[1]
    f32 = jnp.float32

    eba = _EB_A
    nblk_a = e // eba
    l3 = L_e.reshape(nblk_a, 1, eba)
    w3 = W_e.reshape(nblk_a, 1, eba)
    spikes_t = pl.pallas_call(
        _spikes_kernel,
        grid=(nblk_a,),
        in_specs=[
            pl.BlockSpec((b, eba), lambda i: (0, i)),
            pl.BlockSpec((b, eba), lambda i: (0, i)),
            pl.BlockSpec((1, 1, eba), lambda i: (i, 0, 0)),
            pl.BlockSpec((1, 1, eba), lambda i: (i, 0, 0)),
        ],
        out_specs=pl.BlockSpec((eba, b), lambda i: (i, 0)),
        out_shape=jax.ShapeDtypeStruct((e, b), f32),
    )(syn_travel, syn_value, l3, w3)

    ce = 896
    tgt3 = tgt.reshape(e // _IR, _IR)
    isynt = _make_sc_scatter(b, e, n, ce)(spikes_t, tgt3)

    tail_cols = n + no + 3
    tail_pad = _EB_C - tail_cols
    vexc, vexct, tail = pl.pallas_call(
        functools.partial(_neuron_kernel, nh, no, tail_pad),
        out_shape=[
            jax.ShapeDtypeStruct((b, n), f32),
            jax.ShapeDtypeStruct((n, b), f32),
            jax.ShapeDtypeStruct((b, _EB_C), f32),
        ],
    )(isynt, vm, acc, input_current, phase.reshape(b, 1))

    src3 = src.reshape(e // _IR, _IR)
    gathered_t = _make_sc_gather(b, e, n, ce)(vexct, src3)

    ebc = _EB_C
    nblk_c = e // ebc
    out_cols = 2 * e + tail_cols
    l3c = L_e.reshape(nblk_c, 1, ebc)

    def edge_map2(i):
        j = jnp.where(i < nblk_c, i, i - nblk_c)
        return (0, jnp.minimum(j, nblk_c - 1))

    def edge_map2t(i):
        j = jnp.where(i < nblk_c, i, i - nblk_c)
        return (jnp.minimum(j, nblk_c - 1), 0)

    def edge_map3(i):
        j = jnp.where(i < nblk_c, i, i - nblk_c)
        return (jnp.minimum(j, nblk_c - 1), 0, 0)

    out = pl.pallas_call(
        functools.partial(_pass2_kernel, nblk_c),
        grid=(2 * nblk_c + 1,),
        in_specs=[
            pl.BlockSpec((b, ebc), edge_map2),
            pl.BlockSpec((b, ebc), edge_map2),
            pl.BlockSpec((1, 1, ebc), edge_map3),
            pl.BlockSpec((ebc, b), edge_map2t),
            pl.BlockSpec((b, ebc), lambda i: (0, 0)),
        ],
        out_specs=pl.BlockSpec((b, ebc), lambda i: (0, i)),
        out_shape=jax.ShapeDtypeStruct((b, out_cols), f32),
    )(syn_travel, syn_value, l3c, gathered_t, tail)

    return out


# speedup vs baseline: 7.0722x; 1.2822x over previous
import functools

import jax
import jax.numpy as jnp
from jax import lax
from jax.experimental import pallas as pl
from jax.experimental.pallas import tpu as pltpu
from jax.experimental.pallas import tpu_sc as plsc

_TAU = 10.0
_DT = 1.0
_THRESH = 0.5
_VMAX = 1.0
_ATOL = 1e-5
_RTOL = 1e-8

_EB_A = 2048
_EB_C = 3072
_PB = 128

_NC = 2
_NS = 16
_NW = _NC * _NS
_IR = 128


def _mesh():
    return plsc.VectorSubcoreMesh(
        core_axis_name="c", subcore_axis_name="s",
        num_cores=_NC, num_subcores=_NS)


def _sc_params():
    return pltpu.CompilerParams(needs_layout_passes=False,
                                use_tc_tiling_on_sc=False)


def _spikes_kernel(st_lo, sv_lo, l_lo, w_lo, st_hi, sv_hi, l_hi, w_hi,
                   spkt_ref):
    def spk(st_ref, sv_ref, l_ref, w_ref):
        st = st_ref[...]
        lvals = l_ref[0]
        arrived = jnp.abs(st - lvals) <= (_ATOL + _RTOL * jnp.abs(lvals))
        return jnp.where(arrived, sv_ref[...] * w_ref[0], 0.0)

    lo = spk(st_lo, sv_lo, l_lo, w_lo)
    hi = spk(st_hi, sv_hi, l_hi, w_hi)
    spkt_ref[...] = jnp.concatenate([lo.T, hi.T], axis=1)


def _make_sc_scatter(b, e, n, ce):
    eper = e // _NW
    nchunks = eper // ce
    assert nchunks % 2 == 0
    kr = ce // _IR
    zrows = n // _NS
    zc = 8
    assert zrows % zc == 0

    @functools.partial(
        pl.kernel, mesh=_mesh(),
        out_type=jax.ShapeDtypeStruct((_NC, n, b), jnp.float32),
        scratch_types=[
            pltpu.VMEM((2, kr, _IR, b), jnp.float32),
            pltpu.VMEM((2, kr, _IR), jnp.int32),
            pltpu.VMEM((zc, b), jnp.float32),
            pltpu.VMEM_SHARED((n, b), jnp.float32),
            pltpu.SemaphoreType.DMA,
            pltpu.SemaphoreType.DMA,
            pltpu.SemaphoreType.DMA,
            pltpu.SemaphoreType.DMA,
        ],
        compiler_params=_sc_params(),
    )
    def k(spkt_hbm, tgt3_hbm, isynt_hbm, val_v, idx_v, zbuf, shared,
          in0, in1, sc0, sc1):
        cid = lax.axis_index("c")
        sid = lax.axis_index("s")
        wid = sid * _NC + cid
        rb0 = (wid * eper) // _IR
        insems = (in0, in1)
        scsems = (sc0, sc1)

        for r in range(zc):
            for q in range(b // 16):
                zbuf[r, pl.ds(q * 16, 16)] = jnp.zeros((16,), jnp.float32)
        for z in range(zrows // zc):
            pltpu.sync_copy(
                zbuf, shared.at[pl.ds(sid * zrows + z * zc, zc)])
        plsc.subcore_barrier()

        def start_in(ci, p):
            pltpu.async_copy(spkt_hbm.at[pl.ds(rb0 + ci * kr, kr)],
                             val_v.at[p], insems[p])
            pltpu.async_copy(tgt3_hbm.at[pl.ds(rb0 + ci * kr, kr)],
                             idx_v.at[p], insems[p])

        def wait_in(p):
            pltpu.make_async_copy(spkt_hbm.at[pl.ds(0, kr)], val_v.at[p],
                                  insems[p]).wait()
            pltpu.make_async_copy(tgt3_hbm.at[pl.ds(0, kr)], idx_v.at[p],
                                  insems[p]).wait()

        def issue_scatter(p):
            for j in range(kr):
                pltpu.async_copy(val_v.at[p, j],
                                 shared.at[idx_v.at[p, j]], scsems[p],
                                 add=True)

        def drain_scatter(p):
            for j in range(kr):
                pltpu.make_async_copy(val_v.at[p, j],
                                      shared.at[idx_v.at[p, j]],
                                      scsems[p]).wait()

        start_in(0, 0)

        def pair_body(i, _):
            start_in(2 * i + 1, 1)
            wait_in(0)
            issue_scatter(0)
            drain_scatter(0)

            @pl.when(2 * i + 2 < nchunks)
            def _():
                start_in(2 * i + 2, 0)
            wait_in(1)
            issue_scatter(1)
            drain_scatter(1)
            return 0
        lax.fori_loop(0, nchunks // 2, pair_body, 0)

        plsc.subcore_barrier()

        @pl.when(sid == 0)
        def _():
            pltpu.sync_copy(shared, isynt_hbm.at[cid])

    return k


def _neuron_kernel(nh, no, tail_pad, isynt_ref, vm_ref, acc_ref, inp_ref,
                   phase_ref, vexct_ref, tail_ref):
    inject = (phase_ref[...] == 2).astype(jnp.float32)
    inp = inp_ref[...]
    b = inp.shape[0]
    i_syn = (isynt_ref[0] + isynt_ref[1]).T
    i_inj = jnp.concatenate(
        [inp * inject, jnp.zeros((b, no), jnp.float32)], axis=1)
    i_syn = i_syn + i_inj
    vm = vm_ref[...]
    vm1 = vm + (i_syn - vm) * (_DT / _TAU)
    v_exc = jnp.maximum(0.0, vm1 - _THRESH)
    fired = (v_exc > 0).astype(jnp.float32)
    vm2 = vm1 - vm1 * fired + 0.2 * fired
    acc1 = acc_ref[...] + vm1[:, -no:]
    spike_rate = jnp.mean(fired, axis=1, keepdims=True)
    input_norm = jnp.sqrt(jnp.sum(inp * inp, axis=1, keepdims=True)) * inject
    vexct_ref[...] = v_exc.T
    tail_ref[...] = jnp.concatenate(
        [vm2, acc1, inject, spike_rate, input_norm,
         jnp.zeros((b, tail_pad), jnp.float32)], axis=1)


def _make_sc_gather(b, e, n, ce):
    eper = e // _NW
    nchunks = eper // ce
    assert nchunks % 2 == 0
    kr = ce // _IR
    srows = n // _NS

    @functools.partial(
        pl.kernel, mesh=_mesh(),
        out_type=jax.ShapeDtypeStruct((e // _IR, _IR, b), jnp.float32),
        scratch_types=[
            pltpu.VMEM((2, kr, _IR, b), jnp.float32),
            pltpu.VMEM((2, kr, _IR), jnp.int32),
            pltpu.VMEM_SHARED((n, b), jnp.float32),
            pltpu.SemaphoreType.DMA,
            pltpu.SemaphoreType.DMA,
            pltpu.SemaphoreType.DMA,
            pltpu.SemaphoreType.DMA,
            pltpu.SemaphoreType.DMA,
            pltpu.SemaphoreType.DMA,
        ],
        compiler_params=_sc_params(),
    )
    def k(vexct_hbm, src3_hbm, gt_hbm, g_v, idx_v, shared,
          in0, in1, ga0, ga1, out0, out1):
        cid = lax.axis_index("c")
        sid = lax.axis_index("s")
        wid = sid * _NC + cid
        rb0 = (wid * eper) // _IR
        insems = (in0, in1)
        gasems = (ga0, ga1)
        outsems = (out0, out1)

        pltpu.sync_copy(vexct_hbm.at[pl.ds(sid * srows, srows)],
                        shared.at[pl.ds(sid * srows, srows)])
        plsc.subcore_barrier()

        def start_in(ci, p):
            pltpu.async_copy(src3_hbm.at[pl.ds(rb0 + ci * kr, kr)],
                             idx_v.at[p], insems[p])

        def wait_in(p):
            pltpu.make_async_copy(src3_hbm.at[pl.ds(0, kr)], idx_v.at[p],
                                  insems[p]).wait()

        def issue_gather(p):
            for j in range(kr):
                pltpu.async_copy(shared.at[idx_v.at[p, j]],
                                 g_v.at[p, j], gasems[p])

        def drain_gather(p):
            for j in range(kr):
                pltpu.make_async_copy(shared.at[idx_v.at[p, j]],
                                      g_v.at[p, j], gasems[p]).wait()

        def start_out(ci, p):
            pltpu.async_copy(g_v.at[p],
                             gt_hbm.at[pl.ds(rb0 + ci * kr, kr)],
                             outsems[p])

        def drain_out(p):
            pltpu.make_async_copy(g_v.at[p], gt_hbm.at[pl.ds(0, kr)],
                                  outsems[p]).wait()

        start_in(0, 0)

        def pair_body(i, _):
            start_in(2 * i + 1, 1)
            wait_in(0)

            @pl.when(i > 0)
            def _():
                drain_out(0)
            issue_gather(0)
            drain_gather(0)
            start_out(2 * i, 0)

            @pl.when(2 * i + 2 < nchunks)
            def _():
                start_in(2 * i + 2, 0)
            wait_in(1)

            @pl.when(i > 0)
            def _():
                drain_out(1)
            issue_gather(1)
            drain_gather(1)
            start_out(2 * i + 1, 1)
            return 0
        lax.fori_loop(0, nchunks // 2, pair_body, 0)
        drain_out(0)
        drain_out(1)

    return k


def _pass2_kernel(n_st_blocks, st_ref, sv_ref, l_ref, gt_ref, tail_ref,
                  out_ref, sv_stash):
    i = pl.program_id(0)
    last = 2 * n_st_blocks
    nhalf = n_st_blocks // 2

    @pl.when((i < last) & (i % 2 == 0))
    def _():
        st = st_ref[...]
        sv = sv_ref[...]
        lvals = l_ref[0]
        b = st.shape[0]
        gt = gt_ref[...]
        eb = jnp.minimum(i // 2, n_st_blocks - 1)
        g = jnp.where(eb < nhalf, gt[:, :b], gt[:, b:]).T

        arrived = jnp.abs(st - lvals) <= (_ATOL + _RTOL * jnp.abs(lvals))
        stz = jnp.where(arrived, 0.0, st)
        svz = jnp.where(arrived, 0.0, sv)

        new = (g > 0) & (st == 0)
        st2 = stz + jnp.where(stz > 0, _DT * _VMAX, 0.0) \
                  + jnp.where(new, _DT * _VMAX, 0.0)
        sv2 = svz + jnp.where(new, g, 0.0)
        out_ref[...] = st2
        sv_stash[...] = sv2

    @pl.when((i < last) & (i % 2 == 1))
    def _():
        out_ref[...] = sv_stash[...]

    @pl.when(i == last)
    def _():
        out_ref[...] = tail_ref[...]


def _pack_idx(idx, e):
    return idx.reshape(2, e // _IR, _IR // 2).transpose(1, 2, 0).reshape(
        e // _IR, _IR)


def kernel(syn_travel, syn_value, vm, acc, input_current, L_e, W_e, phase,
           src, tgt):
    b, e = syn_travel.shape
    n = vm.shape[1]
    nh = input_current.shape[1]
    no = acc.shape[1]
    f32 = jnp.float32
    half = e // 2

    eba = _EB_A
    nblk_a = half // eba
    l3 = L_e.reshape(2 * nblk_a, 1, eba)
    w3 = W_e.reshape(2 * nblk_a, 1, eba)

    def lo2(i):
        return (0, i)

    def hi2(i):
        return (0, i + nblk_a)

    def lo3(i):
        return (i, 0, 0)

    def hi3(i):
        return (i + nblk_a, 0, 0)

    spikes_t = pl.pallas_call(
        _spikes_kernel,
        grid=(nblk_a,),
        in_specs=[
            pl.BlockSpec((b, eba), lo2),
            pl.BlockSpec((b, eba), lo2),
            pl.BlockSpec((1, 1, eba), lo3),
            pl.BlockSpec((1, 1, eba), lo3),
            pl.BlockSpec((b, eba), hi2),
            pl.BlockSpec((b, eba), hi2),
            pl.BlockSpec((1, 1, eba), hi3),
            pl.BlockSpec((1, 1, eba), hi3),
        ],
        out_specs=pl.BlockSpec((eba, _PB), lambda i: (i, 0)),
        out_shape=jax.ShapeDtypeStruct((half, _PB), f32),
    )(syn_travel, syn_value, l3, w3, syn_travel, syn_value, l3, w3)

    ce = 384
    isynt = _make_sc_scatter(b, e, n, ce)(
        spikes_t.reshape(e // _IR, _IR, b), _pack_idx(tgt, e))

    tail_cols = n + no + 3
    tail_pad = _EB_C - tail_cols
    vexct, tail = pl.pallas_call(
        functools.partial(_neuron_kernel, nh, no, tail_pad),
        out_shape=[
            jax.ShapeDtypeStruct((n, b), f32),
            jax.ShapeDtypeStruct((b, _EB_C), f32),
        ],
    )(isynt, vm, acc, input_current, phase.reshape(b, 1))

    gathered_t = _make_sc_gather(b, e, n, ce)(
        vexct, _pack_idx(src, e)).reshape(half, _PB)

    ebc = _EB_C
    nblk_c = e // ebc
    nhalf_c = nblk_c // 2
    out_cols = 2 * e + tail_cols
    l3c = L_e.reshape(nblk_c, 1, ebc)

    def eb_of(i):
        return jnp.minimum(i // 2, nblk_c - 1)

    def edge_map2(i):
        return (0, eb_of(i))

    def edge_map2t(i):
        return (eb_of(i) % nhalf_c, 0)

    def edge_map3(i):
        return (eb_of(i), 0, 0)

    def out_map(i):
        return (0, jnp.where(i == 2 * nblk_c, 2 * nblk_c,
                             jnp.where(i % 2 == 0, i // 2,
                                       nblk_c + i // 2)))

    out = pl.pallas_call(
        functools.partial(_pass2_kernel, nblk_c),
        grid=(2 * nblk_c + 1,),
        in_specs=[
            pl.BlockSpec((b, ebc), edge_map2),
            pl.BlockSpec((b, ebc), edge_map2),
            pl.BlockSpec((1, 1, ebc), edge_map3),
            pl.BlockSpec((ebc, _PB), edge_map2t),
            pl.BlockSpec((b, _EB_C), lambda i: (0, 0)),
        ],
        out_specs=pl.BlockSpec((b, ebc), out_map),
        out_shape=jax.ShapeDtypeStruct((b, out_cols), f32),
        scratch_shapes=[pltpu.VMEM((b, ebc), f32)],
    )(syn_travel, syn_value, l3c, gathered_t, tail)

    return out


# speedup vs baseline: 8.2394x; 1.1650x over previous
import functools

import jax
import jax.numpy as jnp
from jax import lax
from jax.experimental import pallas as pl
from jax.experimental.pallas import tpu as pltpu
from jax.experimental.pallas import tpu_sc as plsc

_TAU = 10.0
_DT = 1.0
_THRESH = 0.5
_VMAX = 1.0
_ATOL = 1e-5
_RTOL = 1e-8

_EB_A = 2048
_EB_C = 3072
_PB = 128

_NC = 2
_NS = 16
_NW = _NC * _NS
_IR = 128


def _mesh():
    return plsc.VectorSubcoreMesh(
        core_axis_name="c", subcore_axis_name="s",
        num_cores=_NC, num_subcores=_NS)


def _sc_params():
    return pltpu.CompilerParams(needs_layout_passes=False,
                                use_tc_tiling_on_sc=False)


def _spikes_kernel(st_lo, sv_lo, l_lo, w_lo, st_hi, sv_hi, l_hi, w_hi,
                   spkt_ref):
    def spk(st_ref, sv_ref, l_ref, w_ref):
        st = st_ref[...]
        lvals = l_ref[0]
        arrived = jnp.abs(st - lvals) <= (_ATOL + _RTOL * jnp.abs(lvals))
        return jnp.where(arrived, sv_ref[...] * w_ref[0], 0.0)

    lo = spk(st_lo, sv_lo, l_lo, w_lo)
    hi = spk(st_hi, sv_hi, l_hi, w_hi)
    spkt_ref[...] = jnp.concatenate([lo.T, hi.T], axis=1)


def _make_sc_scatter(b, e, n, ce):
    eper = e // _NW
    nchunks = eper // ce
    assert nchunks % 2 == 0
    kr = ce // _IR
    zrows = n // _NS
    zc = 8
    assert zrows % zc == 0

    @functools.partial(
        pl.kernel, mesh=_mesh(),
        out_type=jax.ShapeDtypeStruct((_NC, n, b), jnp.float32),
        scratch_types=[
            pltpu.VMEM((2, kr, _IR, b), jnp.float32),
            pltpu.VMEM((2, kr, _IR), jnp.int32),
            pltpu.VMEM((zc, b), jnp.float32),
            pltpu.VMEM_SHARED((n, b), jnp.float32),
            pltpu.SemaphoreType.DMA,
            pltpu.SemaphoreType.DMA,
            pltpu.SemaphoreType.DMA,
            pltpu.SemaphoreType.DMA,
        ],
        compiler_params=_sc_params(),
    )
    def k(spkt_hbm, tgt3_hbm, isynt_hbm, val_v, idx_v, zbuf, shared,
          in0, in1, sc0, sc1):
        cid = lax.axis_index("c")
        sid = lax.axis_index("s")
        wid = sid * _NC + cid
        rb0 = (wid * eper) // _IR
        insems = (in0, in1)
        scsems = (sc0, sc1)

        for r in range(zc):
            for q in range(b // 16):
                zbuf[r, pl.ds(q * 16, 16)] = jnp.zeros((16,), jnp.float32)
        for z in range(zrows // zc):
            pltpu.sync_copy(
                zbuf, shared.at[pl.ds(sid * zrows + z * zc, zc)])
        plsc.subcore_barrier()

        def start_in(ci, p):
            pltpu.async_copy(spkt_hbm.at[pl.ds(rb0 + ci * kr, kr)],
                             val_v.at[p], insems[p])
            pltpu.async_copy(tgt3_hbm.at[pl.ds(rb0 + ci * kr, kr)],
                             idx_v.at[p], insems[p])

        def wait_in(p):
            pltpu.make_async_copy(spkt_hbm.at[pl.ds(0, kr)], val_v.at[p],
                                  insems[p]).wait()
            pltpu.make_async_copy(tgt3_hbm.at[pl.ds(0, kr)], idx_v.at[p],
                                  insems[p]).wait()

        def issue_scatter(p):
            for j in range(kr):
                pltpu.async_copy(val_v.at[p, j],
                                 shared.at[idx_v.at[p, j]], scsems[p],
                                 add=True)

        def drain_scatter(p):
            for j in range(kr):
                pltpu.make_async_copy(val_v.at[p, j],
                                      shared.at[idx_v.at[p, j]],
                                      scsems[p]).wait()

        start_in(0, 0)

        def pair_body(i, _):
            start_in(2 * i + 1, 1)
            wait_in(0)
            issue_scatter(0)
            drain_scatter(0)

            @pl.when(2 * i + 2 < nchunks)
            def _():
                start_in(2 * i + 2, 0)
            wait_in(1)
            issue_scatter(1)
            drain_scatter(1)
            return 0
        lax.fori_loop(0, nchunks // 2, pair_body, 0)

        plsc.subcore_barrier()

        @pl.when(sid == 0)
        def _():
            pltpu.sync_copy(shared, isynt_hbm.at[cid])

    return k


def _neuron_kernel(nh, no, tail_pad, isynt_ref, vm_ref, acc_ref, inp_ref,
                   phase_ref, vexct_ref, tail_ref):
    inject = (phase_ref[...] == 2).astype(jnp.float32)
    inp = inp_ref[...]
    b = inp.shape[0]
    i_syn = (isynt_ref[0] + isynt_ref[1]).T
    i_inj = jnp.concatenate(
        [inp * inject, jnp.zeros((b, no), jnp.float32)], axis=1)
    i_syn = i_syn + i_inj
    vm = vm_ref[...]
    vm1 = vm + (i_syn - vm) * (_DT / _TAU)
    v_exc = jnp.maximum(0.0, vm1 - _THRESH)
    fired = (v_exc > 0).astype(jnp.float32)
    vm2 = vm1 - vm1 * fired + 0.2 * fired
    acc1 = acc_ref[...] + vm1[:, -no:]
    spike_rate = jnp.mean(fired, axis=1, keepdims=True)
    input_norm = jnp.sqrt(jnp.sum(inp * inp, axis=1, keepdims=True)) * inject
    vexct_ref[...] = v_exc.T
    tail_ref[...] = jnp.concatenate(
        [vm2, acc1, inject, spike_rate, input_norm,
         jnp.zeros((b, tail_pad), jnp.float32)], axis=1)


def _make_sc_gather(b, e, n, ce):
    eper = e // _NW
    nchunks = eper // ce
    assert nchunks % 2 == 0
    kr = ce // _IR
    srows = n // _NS

    @functools.partial(
        pl.kernel, mesh=_mesh(),
        out_type=jax.ShapeDtypeStruct((e // _IR, _IR, b), jnp.float32),
        scratch_types=[
            pltpu.VMEM((2, kr, _IR, b), jnp.float32),
            pltpu.VMEM((2, kr, _IR), jnp.int32),
            pltpu.VMEM_SHARED((n, b), jnp.float32),
            pltpu.SemaphoreType.DMA,
            pltpu.SemaphoreType.DMA,
            pltpu.SemaphoreType.DMA,
            pltpu.SemaphoreType.DMA,
            pltpu.SemaphoreType.DMA,
            pltpu.SemaphoreType.DMA,
        ],
        compiler_params=_sc_params(),
    )
    def k(vexct_hbm, src3_hbm, gt_hbm, g_v, idx_v, shared,
          in0, in1, ga0, ga1, out0, out1):
        cid = lax.axis_index("c")
        sid = lax.axis_index("s")
        wid = sid * _NC + cid
        rb0 = (wid * eper) // _IR
        insems = (in0, in1)
        gasems = (ga0, ga1)
        outsems = (out0, out1)

        pltpu.sync_copy(vexct_hbm.at[pl.ds(sid * srows, srows)],
                        shared.at[pl.ds(sid * srows, srows)])
        plsc.subcore_barrier()

        def start_in(ci, p):
            pltpu.async_copy(src3_hbm.at[pl.ds(rb0 + ci * kr, kr)],
                             idx_v.at[p], insems[p])

        def wait_in(p):
            pltpu.make_async_copy(src3_hbm.at[pl.ds(0, kr)], idx_v.at[p],
                                  insems[p]).wait()

        def issue_gather(p):
            for j in range(kr):
                pltpu.async_copy(shared.at[idx_v.at[p, j]],
                                 g_v.at[p, j], gasems[p])

        def drain_gather(p):
            for j in range(kr):
                pltpu.make_async_copy(shared.at[idx_v.at[p, j]],
                                      g_v.at[p, j], gasems[p]).wait()

        def start_out(ci, p):
            pltpu.async_copy(g_v.at[p],
                             gt_hbm.at[pl.ds(rb0 + ci * kr, kr)],
                             outsems[p])

        def drain_out(p):
            pltpu.make_async_copy(g_v.at[p], gt_hbm.at[pl.ds(0, kr)],
                                  outsems[p]).wait()

        start_in(0, 0)

        def pair_body(i, _):
            start_in(2 * i + 1, 1)
            wait_in(0)

            @pl.when(i > 0)
            def _():
                drain_out(0)
            issue_gather(0)
            drain_gather(0)
            start_out(2 * i, 0)

            @pl.when(2 * i + 2 < nchunks)
            def _():
                start_in(2 * i + 2, 0)
            wait_in(1)

            @pl.when(i > 0)
            def _():
                drain_out(1)
            issue_gather(1)
            drain_gather(1)
            start_out(2 * i + 1, 1)
            return 0
        lax.fori_loop(0, nchunks // 2, pair_body, 0)
        drain_out(0)
        drain_out(1)

    return k


def _pass2_kernel(n_st_blocks, st_lo_ref, sv_lo_ref, l_lo_ref, st_hi_ref,
                  sv_hi_ref, l_hi_ref, gt_ref, tail_ref, out_ref,
                  stash_sthi, stash_svlo, stash_svhi):
    i = pl.program_id(0)
    last = 2 * n_st_blocks

    def half(st_ref, sv_ref, l_ref, g):
        st = st_ref[...]
        sv = sv_ref[...]
        lvals = l_ref[0]
        arrived = jnp.abs(st - lvals) <= (_ATOL + _RTOL * jnp.abs(lvals))
        stz = jnp.where(arrived, 0.0, st)
        svz = jnp.where(arrived, 0.0, sv)
        new = (g > 0) & (st == 0)
        st2 = stz + jnp.where(stz > 0, _DT * _VMAX, 0.0) \
                  + jnp.where(new, _DT * _VMAX, 0.0)
        sv2 = svz + jnp.where(new, g, 0.0)
        return st2, sv2

    @pl.when((i < last) & (i % 4 == 0))
    def _():
        b = st_lo_ref.shape[0]
        gt = gt_ref[...]
        st2_lo, sv2_lo = half(st_lo_ref, sv_lo_ref, l_lo_ref, gt[:, :b].T)
        st2_hi, sv2_hi = half(st_hi_ref, sv_hi_ref, l_hi_ref, gt[:, b:].T)
        out_ref[...] = st2_lo
        stash_sthi[...] = st2_hi
        stash_svlo[...] = sv2_lo
        stash_svhi[...] = sv2_hi

    @pl.when((i < last) & (i % 4 == 1))
    def _():
        out_ref[...] = stash_sthi[...]

    @pl.when((i < last) & (i % 4 == 2))
    def _():
        out_ref[...] = stash_svlo[...]

    @pl.when((i < last) & (i % 4 == 3))
    def _():
        out_ref[...] = stash_svhi[...]

    @pl.when(i == last)
    def _():
        out_ref[...] = tail_ref[...]


def _pack_idx(idx, e):
    return idx.reshape(2, e // _IR, _IR // 2).transpose(1, 2, 0).reshape(
        e // _IR, _IR)


def kernel(syn_travel, syn_value, vm, acc, input_current, L_e, W_e, phase,
           src, tgt):
    b, e = syn_travel.shape
    n = vm.shape[1]
    nh = input_current.shape[1]
    no = acc.shape[1]
    f32 = jnp.float32
    half = e // 2

    eba = _EB_A
    nblk_a = half // eba
    l3 = L_e.reshape(2 * nblk_a, 1, eba)
    w3 = W_e.reshape(2 * nblk_a, 1, eba)

    def lo2(i):
        return (0, i)

    def hi2(i):
        return (0, i + nblk_a)

    def lo3(i):
        return (i, 0, 0)

    def hi3(i):
        return (i + nblk_a, 0, 0)

    spikes_t = pl.pallas_call(
        _spikes_kernel,
        grid=(nblk_a,),
        in_specs=[
            pl.BlockSpec((b, eba), lo2),
            pl.BlockSpec((b, eba), lo2),
            pl.BlockSpec((1, 1, eba), lo3),
            pl.BlockSpec((1, 1, eba), lo3),
            pl.BlockSpec((b, eba), hi2),
            pl.BlockSpec((b, eba), hi2),
            pl.BlockSpec((1, 1, eba), hi3),
            pl.BlockSpec((1, 1, eba), hi3),
        ],
        out_specs=pl.BlockSpec((eba, _PB), lambda i: (i, 0)),
        out_shape=jax.ShapeDtypeStruct((half, _PB), f32),
    )(syn_travel, syn_value, l3, w3, syn_travel, syn_value, l3, w3)

    ce = 384
    isynt = _make_sc_scatter(b, e, n, ce)(
        spikes_t.reshape(e // _IR, _IR, b), _pack_idx(tgt, e))

    tail_cols = n + no + 3
    tail_pad = _EB_C - tail_cols
    vexct, tail = pl.pallas_call(
        functools.partial(_neuron_kernel, nh, no, tail_pad),
        out_shape=[
            jax.ShapeDtypeStruct((n, b), f32),
            jax.ShapeDtypeStruct((b, _EB_C), f32),
        ],
    )(isynt, vm, acc, input_current, phase.reshape(b, 1))

    gathered_t = _make_sc_gather(b, e, n, ce)(
        vexct, _pack_idx(src, e)).reshape(half, _PB)

    ebc = _EB_C
    nblk_c = e // ebc
    nhalf_c = nblk_c // 2
    out_cols = 2 * e + tail_cols
    l3c = L_e.reshape(nblk_c, 1, ebc)

    def q_of(i):
        return jnp.minimum(i // 4, nhalf_c - 1)

    def lo_map2(i):
        return (0, q_of(i))

    def hi_map2(i):
        return (0, nhalf_c + q_of(i))

    def lo_map3(i):
        return (q_of(i), 0, 0)

    def hi_map3(i):
        return (nhalf_c + q_of(i), 0, 0)

    def gt_map(i):
        return (q_of(i), 0)

    def out_map(i):
        return (0, jnp.where(i == 2 * nblk_c, 2 * nblk_c,
                             (i % 4) * nhalf_c + i // 4))

    out = pl.pallas_call(
        functools.partial(_pass2_kernel, nblk_c),
        grid=(2 * nblk_c + 1,),
        in_specs=[
            pl.BlockSpec((b, ebc), lo_map2),
            pl.BlockSpec((b, ebc), lo_map2),
            pl.BlockSpec((1, 1, ebc), lo_map3),
            pl.BlockSpec((b, ebc), hi_map2),
            pl.BlockSpec((b, ebc), hi_map2),
            pl.BlockSpec((1, 1, ebc), hi_map3),
            pl.BlockSpec((ebc, _PB), gt_map),
            pl.BlockSpec((b, _EB_C), lambda i: (0, 0)),
        ],
        out_specs=pl.BlockSpec((b, ebc), out_map),
        out_shape=jax.ShapeDtypeStruct((b, out_cols), f32),
        scratch_shapes=[pltpu.VMEM((b, ebc), f32),
                        pltpu.VMEM((b, ebc), f32),
                        pltpu.VMEM((b, ebc), f32)],
    )(syn_travel, syn_value, l3c, syn_travel, syn_value, l3c,
      gathered_t, tail)

    return out


# speedup vs baseline: 9.4501x; 1.1469x over previous
import functools

import jax
import jax.numpy as jnp
from jax import lax
from jax.experimental import pallas as pl
from jax.experimental.pallas import tpu as pltpu
from jax.experimental.pallas import tpu_sc as plsc

_TAU = 10.0
_DT = 1.0
_THRESH = 0.5
_VMAX = 1.0
_ATOL = 1e-5
_RTOL = 1e-8

_EB_A = 4096
_EB_C = 6144
_PB = 128

_NC = 2
_NS = 16
_NW = _NC * _NS
_IR = 128


def _mesh():
    return plsc.VectorSubcoreMesh(
        core_axis_name="c", subcore_axis_name="s",
        num_cores=_NC, num_subcores=_NS)


def _sc_params():
    return pltpu.CompilerParams(needs_layout_passes=False,
                                use_tc_tiling_on_sc=False)


def _spikes_kernel(st_lo, sv_lo, l_lo, w_lo, st_hi, sv_hi, l_hi, w_hi,
                   spkt_ref):
    def spk(st_ref, sv_ref, l_ref, w_ref):
        st = st_ref[...]
        lvals = l_ref[0]
        arrived = jnp.abs(st - lvals) <= (_ATOL + _RTOL * jnp.abs(lvals))
        return jnp.where(arrived, sv_ref[...] * w_ref[0], 0.0)

    lo = spk(st_lo, sv_lo, l_lo, w_lo)
    hi = spk(st_hi, sv_hi, l_hi, w_hi)
    spkt_ref[...] = jnp.concatenate([lo.T, hi.T], axis=1)


def _make_sc_scatter(b, e, n, ce):
    eper = e // _NW
    nchunks = eper // ce
    assert nchunks % 2 == 0
    kr = ce // _IR
    zrows = n // _NS
    zc = 8
    assert zrows % zc == 0

    @functools.partial(
        pl.kernel, mesh=_mesh(),
        out_type=jax.ShapeDtypeStruct((_NC, n, b), jnp.float32),
        scratch_types=[
            pltpu.VMEM((2, kr, _IR, b), jnp.float32),
            pltpu.VMEM((2, kr, _IR), jnp.int32),
            pltpu.VMEM((zc, b), jnp.float32),
            pltpu.VMEM_SHARED((n, b), jnp.float32),
            pltpu.SemaphoreType.DMA,
            pltpu.SemaphoreType.DMA,
            pltpu.SemaphoreType.DMA,
            pltpu.SemaphoreType.DMA,
        ],
        compiler_params=_sc_params(),
    )
    def k(spkt_hbm, tgt3_hbm, isynt_hbm, val_v, idx_v, zbuf, shared,
          in0, in1, sc0, sc1):
        cid = lax.axis_index("c")
        sid = lax.axis_index("s")
        wid = sid * _NC + cid
        rb0 = (wid * eper) // _IR
        insems = (in0, in1)
        scsems = (sc0, sc1)

        for r in range(zc):
            for q in range(b // 16):
                zbuf[r, pl.ds(q * 16, 16)] = jnp.zeros((16,), jnp.float32)
        for z in range(zrows // zc):
            pltpu.sync_copy(
                zbuf, shared.at[pl.ds(sid * zrows + z * zc, zc)])
        plsc.subcore_barrier()

        def start_in(ci, p):
            pltpu.async_copy(spkt_hbm.at[pl.ds(rb0 + ci * kr, kr)],
                             val_v.at[p], insems[p])
            pltpu.async_copy(tgt3_hbm.at[pl.ds(rb0 + ci * kr, kr)],
                             idx_v.at[p], insems[p])

        def wait_in(p):
            pltpu.make_async_copy(spkt_hbm.at[pl.ds(0, kr)], val_v.at[p],
                                  insems[p]).wait()
            pltpu.make_async_copy(tgt3_hbm.at[pl.ds(0, kr)], idx_v.at[p],
                                  insems[p]).wait()

        def issue_scatter(p):
            for j in range(kr):
                pltpu.async_copy(val_v.at[p, j],
                                 shared.at[idx_v.at[p, j]], scsems[p],
                                 add=True)

        def drain_scatter(p):
            for j in range(kr):
                pltpu.make_async_copy(val_v.at[p, j],
                                      shared.at[idx_v.at[p, j]],
                                      scsems[p]).wait()

        start_in(0, 0)

        def pair_body(i, _):
            start_in(2 * i + 1, 1)
            wait_in(0)
            issue_scatter(0)
            drain_scatter(0)

            @pl.when(2 * i + 2 < nchunks)
            def _():
                start_in(2 * i + 2, 0)
            wait_in(1)
            issue_scatter(1)
            drain_scatter(1)
            return 0
        lax.fori_loop(0, nchunks // 2, pair_body, 0)

        plsc.subcore_barrier()

        @pl.when(sid == 0)
        def _():
            pltpu.sync_copy(shared, isynt_hbm.at[cid])

    return k


def _neuron_kernel(nh, no, tail_pad, isynt_ref, vm_ref, acc_ref, inp_ref,
                   phase_ref, vexct_ref, tail_ref):
    inject = (phase_ref[...] == 2).astype(jnp.float32)
    inp = inp_ref[...]
    b = inp.shape[0]
    i_syn = (isynt_ref[0] + isynt_ref[1]).T
    i_inj = jnp.concatenate(
        [inp * inject, jnp.zeros((b, no), jnp.float32)], axis=1)
    i_syn = i_syn + i_inj
    vm = vm_ref[...]
    vm1 = vm + (i_syn - vm) * (_DT / _TAU)
    v_exc = jnp.maximum(0.0, vm1 - _THRESH)
    fired = (v_exc > 0).astype(jnp.float32)
    vm2 = vm1 - vm1 * fired + 0.2 * fired
    acc1 = acc_ref[...] + vm1[:, -no:]
    spike_rate = jnp.mean(fired, axis=1, keepdims=True)
    input_norm = jnp.sqrt(jnp.sum(inp * inp, axis=1, keepdims=True)) * inject
    vexct_ref[...] = v_exc.T
    tail_ref[...] = jnp.concatenate(
        [vm2, acc1, inject, spike_rate, input_norm,
         jnp.zeros((b, tail_pad), jnp.float32)], axis=1)


def _make_sc_gather(b, e, n, ce):
    eper = e // _NW
    nchunks = eper // ce
    assert nchunks % 2 == 0
    kr = ce // _IR
    srows = n // _NS

    @functools.partial(
        pl.kernel, mesh=_mesh(),
        out_type=jax.ShapeDtypeStruct((e // _IR, _IR, b), jnp.float32),
        scratch_types=[
            pltpu.VMEM((2, kr, _IR, b), jnp.float32),
            pltpu.VMEM((2, kr, _IR), jnp.int32),
            pltpu.VMEM_SHARED((n, b), jnp.float32),
            pltpu.SemaphoreType.DMA,
            pltpu.SemaphoreType.DMA,
            pltpu.SemaphoreType.DMA,
            pltpu.SemaphoreType.DMA,
            pltpu.SemaphoreType.DMA,
            pltpu.SemaphoreType.DMA,
        ],
        compiler_params=_sc_params(),
    )
    def k(vexct_hbm, src3_hbm, gt_hbm, g_v, idx_v, shared,
          in0, in1, ga0, ga1, out0, out1):
        cid = lax.axis_index("c")
        sid = lax.axis_index("s")
        wid = sid * _NC + cid
        rb0 = (wid * eper) // _IR
        insems = (in0, in1)
        gasems = (ga0, ga1)
        outsems = (out0, out1)

        pltpu.sync_copy(vexct_hbm.at[pl.ds(sid * srows, srows)],
                        shared.at[pl.ds(sid * srows, srows)])
        plsc.subcore_barrier()

        def start_in(ci, p):
            pltpu.async_copy(src3_hbm.at[pl.ds(rb0 + ci * kr, kr)],
                             idx_v.at[p], insems[p])

        def wait_in(p):
            pltpu.make_async_copy(src3_hbm.at[pl.ds(0, kr)], idx_v.at[p],
                                  insems[p]).wait()

        def issue_gather(p):
            for j in range(kr):
                pltpu.async_copy(shared.at[idx_v.at[p, j]],
                                 g_v.at[p, j], gasems[p])

        def drain_gather(p):
            for j in range(kr):
                pltpu.make_async_copy(shared.at[idx_v.at[p, j]],
                                      g_v.at[p, j], gasems[p]).wait()

        def start_out(ci, p):
            pltpu.async_copy(g_v.at[p],
                             gt_hbm.at[pl.ds(rb0 + ci * kr, kr)],
                             outsems[p])

        def drain_out(p):
            pltpu.make_async_copy(g_v.at[p], gt_hbm.at[pl.ds(0, kr)],
                                  outsems[p]).wait()

        start_in(0, 0)

        def pair_body(i, _):
            start_in(2 * i + 1, 1)
            wait_in(0)

            @pl.when(i > 0)
            def _():
                drain_out(0)
            issue_gather(0)
            drain_gather(0)
            start_out(2 * i, 0)

            @pl.when(2 * i + 2 < nchunks)
            def _():
                start_in(2 * i + 2, 0)
            wait_in(1)

            @pl.when(i > 0)
            def _():
                drain_out(1)
            issue_gather(1)
            drain_gather(1)
            start_out(2 * i + 1, 1)
            return 0
        lax.fori_loop(0, nchunks // 2, pair_body, 0)
        drain_out(0)
        drain_out(1)

    return k


def _pass2_kernel(n_st_blocks, st_lo_ref, sv_lo_ref, l_lo_ref, st_hi_ref,
                  sv_hi_ref, l_hi_ref, gt_ref, tail_ref, out_ref,
                  stash_sthi, stash_svlo, stash_svhi):
    i = pl.program_id(0)
    last = 2 * n_st_blocks

    def half(st_ref, sv_ref, l_ref, g):
        st = st_ref[...]
        sv = sv_ref[...]
        lvals = l_ref[0]
        arrived = jnp.abs(st - lvals) <= (_ATOL + _RTOL * jnp.abs(lvals))
        stz = jnp.where(arrived, 0.0, st)
        svz = jnp.where(arrived, 0.0, sv)
        new = (g > 0) & (st == 0)
        st2 = stz + jnp.where(stz > 0, _DT * _VMAX, 0.0) \
                  + jnp.where(new, _DT * _VMAX, 0.0)
        sv2 = svz + jnp.where(new, g, 0.0)
        return st2, sv2

    @pl.when((i < last) & (i % 4 == 0))
    def _():
        b = st_lo_ref.shape[0]
        gt = gt_ref[...]
        st2_lo, sv2_lo = half(st_lo_ref, sv_lo_ref, l_lo_ref, gt[:, :b].T)
        st2_hi, sv2_hi = half(st_hi_ref, sv_hi_ref, l_hi_ref, gt[:, b:].T)
        out_ref[...] = st2_lo
        stash_sthi[...] = st2_hi
        stash_svlo[...] = sv2_lo
        stash_svhi[...] = sv2_hi

    @pl.when((i < last) & (i % 4 == 1))
    def _():
        out_ref[...] = stash_sthi[...]

    @pl.when((i < last) & (i % 4 == 2))
    def _():
        out_ref[...] = stash_svlo[...]

    @pl.when((i < last) & (i % 4 == 3))
    def _():
        out_ref[...] = stash_svhi[...]

    @pl.when(i == last)
    def _():
        out_ref[...] = tail_ref[...]


def _pack_idx(idx, e):
    return idx.reshape(2, e // _IR, _IR // 2).transpose(1, 2, 0).reshape(
        e // _IR, _IR)


def kernel(syn_travel, syn_value, vm, acc, input_current, L_e, W_e, phase,
           src, tgt):
    b, e = syn_travel.shape
    n = vm.shape[1]
    nh = input_current.shape[1]
    no = acc.shape[1]
    f32 = jnp.float32
    half = e // 2

    eba = _EB_A
    nblk_a = half // eba
    l3 = L_e.reshape(2 * nblk_a, 1, eba)
    w3 = W_e.reshape(2 * nblk_a, 1, eba)

    def lo2(i):
        return (0, i)

    def hi2(i):
        return (0, i + nblk_a)

    def lo3(i):
        return (i, 0, 0)

    def hi3(i):
        return (i + nblk_a, 0, 0)

    spikes_t = pl.pallas_call(
        _spikes_kernel,
        grid=(nblk_a,),
        in_specs=[
            pl.BlockSpec((b, eba), lo2),
            pl.BlockSpec((b, eba), lo2),
            pl.BlockSpec((1, 1, eba), lo3),
            pl.BlockSpec((1, 1, eba), lo3),
            pl.BlockSpec((b, eba), hi2),
            pl.BlockSpec((b, eba), hi2),
            pl.BlockSpec((1, 1, eba), hi3),
            pl.BlockSpec((1, 1, eba), hi3),
        ],
        out_specs=pl.BlockSpec((eba, _PB), lambda i: (i, 0)),
        out_shape=jax.ShapeDtypeStruct((half, _PB), f32),
    )(syn_travel, syn_value, l3, w3, syn_travel, syn_value, l3, w3)

    ce = 384
    isynt = _make_sc_scatter(b, e, n, ce)(
        spikes_t.reshape(e // _IR, _IR, b), _pack_idx(tgt, e))

    tail_cols = n + no + 3
    tail_pad = _EB_C - tail_cols
    vexct, tail = pl.pallas_call(
        functools.partial(_neuron_kernel, nh, no, tail_pad),
        out_shape=[
            jax.ShapeDtypeStruct((n, b), f32),
            jax.ShapeDtypeStruct((b, _EB_C), f32),
        ],
    )(isynt, vm, acc, input_current, phase.reshape(b, 1))

    gathered_t = _make_sc_gather(b, e, n, ce)(
        vexct, _pack_idx(src, e)).reshape(half, _PB)

    ebc = _EB_C
    nblk_c = e // ebc
    nhalf_c = nblk_c // 2
    out_cols = 2 * e + tail_cols
    l3c = L_e.reshape(nblk_c, 1, ebc)

    def q_of(i):
        return jnp.minimum(i // 4, nhalf_c - 1)

    def lo_map2(i):
        return (0, q_of(i))

    def hi_map2(i):
        return (0, nhalf_c + q_of(i))

    def lo_map3(i):
        return (q_of(i), 0, 0)

    def hi_map3(i):
        return (nhalf_c + q_of(i), 0, 0)

    def gt_map(i):
        return (q_of(i), 0)

    def out_map(i):
        return (0, jnp.where(i == 2 * nblk_c, 2 * nblk_c,
                             (i % 4) * nhalf_c + i // 4))

    out = pl.pallas_call(
        functools.partial(_pass2_kernel, nblk_c),
        grid=(2 * nblk_c + 1,),
        in_specs=[
            pl.BlockSpec((b, ebc), lo_map2),
            pl.BlockSpec((b, ebc), lo_map2),
            pl.BlockSpec((1, 1, ebc), lo_map3),
            pl.BlockSpec((b, ebc), hi_map2),
            pl.BlockSpec((b, ebc), hi_map2),
            pl.BlockSpec((1, 1, ebc), hi_map3),
            pl.BlockSpec((ebc, _PB), gt_map),
            pl.BlockSpec((b, _EB_C), lambda i: (0, 0)),
        ],
        out_specs=pl.BlockSpec((b, ebc), out_map),
        out_shape=jax.ShapeDtypeStruct((b, out_cols), f32),
        scratch_shapes=[pltpu.VMEM((b, ebc), f32),
                        pltpu.VMEM((b, ebc), f32),
                        pltpu.VMEM((b, ebc), f32)],
    )(syn_travel, syn_value, l3c, syn_travel, syn_value, l3c,
      gathered_t, tail)

    return out


# speedup vs baseline: 9.6259x; 1.0186x over previous
import functools

import jax
import jax.numpy as jnp
from jax import lax
from jax.experimental import pallas as pl
from jax.experimental.pallas import tpu as pltpu
from jax.experimental.pallas import tpu_sc as plsc

_TAU = 10.0
_DT = 1.0
_THRESH = 0.5
_VMAX = 1.0
_ATOL = 1e-5
_RTOL = 1e-8

_EB_A = 6144
_EB_C = 6144
_PB = 128

_NC = 2
_NS = 16
_NW = _NC * _NS
_IR = 128


def _mesh():
    return plsc.VectorSubcoreMesh(
        core_axis_name="c", subcore_axis_name="s",
        num_cores=_NC, num_subcores=_NS)


def _sc_params():
    return pltpu.CompilerParams(needs_layout_passes=False,
                                use_tc_tiling_on_sc=False)


def _spikes_kernel(st_lo, sv_lo, l_lo, w_lo, st_hi, sv_hi, l_hi, w_hi,
                   spkt_ref):
    def spk(st_ref, sv_ref, l_ref, w_ref):
        st = st_ref[...]
        lvals = l_ref[0]
        arrived = jnp.abs(st - lvals) <= (_ATOL + _RTOL * jnp.abs(lvals))
        return jnp.where(arrived, sv_ref[...] * w_ref[0], 0.0)

    lo = spk(st_lo, sv_lo, l_lo, w_lo)
    hi = spk(st_hi, sv_hi, l_hi, w_hi)
    spkt_ref[...] = jnp.concatenate([lo.T, hi.T], axis=1)


def _make_sc_scatter(b, e, n, ce):
    eper = e // _NW
    nchunks = eper // ce
    assert nchunks % 2 == 0
    kr = ce // _IR
    zrows = n // _NS
    zc = 8
    assert zrows % zc == 0

    @functools.partial(
        pl.kernel, mesh=_mesh(),
        out_type=jax.ShapeDtypeStruct((_NC, n, b), jnp.float32),
        scratch_types=[
            pltpu.VMEM((2, kr, _IR, b), jnp.float32),
            pltpu.VMEM((2, kr, _IR), jnp.int32),
            pltpu.VMEM((zc, b), jnp.float32),
            pltpu.VMEM_SHARED((n, b), jnp.float32),
            pltpu.SemaphoreType.DMA,
            pltpu.SemaphoreType.DMA,
            pltpu.SemaphoreType.DMA,
            pltpu.SemaphoreType.DMA,
        ],
        compiler_params=_sc_params(),
    )
    def k(spkt_hbm, tgt3_hbm, isynt_hbm, val_v, idx_v, zbuf, shared,
          in0, in1, sc0, sc1):
        cid = lax.axis_index("c")
        sid = lax.axis_index("s")
        wid = sid * _NC + cid
        rb0 = (wid * eper) // _IR
        insems = (in0, in1)
        scsems = (sc0, sc1)

        for r in range(zc):
            for q in range(b // 16):
                zbuf[r, pl.ds(q * 16, 16)] = jnp.zeros((16,), jnp.float32)
        for z in range(zrows // zc):
            pltpu.sync_copy(
                zbuf, shared.at[pl.ds(sid * zrows + z * zc, zc)])
        plsc.subcore_barrier()

        def start_in(ci, p):
            pltpu.async_copy(spkt_hbm.at[pl.ds(rb0 + ci * kr, kr)],
                             val_v.at[p], insems[p])
            pltpu.async_copy(tgt3_hbm.at[pl.ds(rb0 + ci * kr, kr)],
                             idx_v.at[p], insems[p])

        def wait_in(p):
            pltpu.make_async_copy(spkt_hbm.at[pl.ds(0, kr)], val_v.at[p],
                                  insems[p]).wait()
            pltpu.make_async_copy(tgt3_hbm.at[pl.ds(0, kr)], idx_v.at[p],
                                  insems[p]).wait()

        def issue_scatter(p):
            for j in range(kr):
                pltpu.async_copy(val_v.at[p, j],
                                 shared.at[idx_v.at[p, j]], scsems[p],
                                 add=True)

        def drain_scatter(p):
            for j in range(kr):
                pltpu.make_async_copy(val_v.at[p, j],
                                      shared.at[idx_v.at[p, j]],
                                      scsems[p]).wait()

        start_in(0, 0)

        def pair_body(i, _):
            start_in(2 * i + 1, 1)
            wait_in(0)
            issue_scatter(0)
            drain_scatter(0)

            @pl.when(2 * i + 2 < nchunks)
            def _():
                start_in(2 * i + 2, 0)
            wait_in(1)
            issue_scatter(1)
            drain_scatter(1)
            return 0
        lax.fori_loop(0, nchunks // 2, pair_body, 0)

        plsc.subcore_barrier()

        @pl.when(sid == 0)
        def _():
            pltpu.sync_copy(shared, isynt_hbm.at[cid])

    return k


def _neuron_kernel(nh, no, tail_pad, isynt_ref, vm_ref, acc_ref, inp_ref,
                   phase_ref, vexct_ref, tail_ref):
    inject = (phase_ref[...] == 2).astype(jnp.float32)
    inp = inp_ref[...]
    b = inp.shape[0]
    i_syn = (isynt_ref[0] + isynt_ref[1]).T
    i_inj = jnp.concatenate(
        [inp * inject, jnp.zeros((b, no), jnp.float32)], axis=1)
    i_syn = i_syn + i_inj
    vm = vm_ref[...]
    vm1 = vm + (i_syn - vm) * (_DT / _TAU)
    v_exc = jnp.maximum(0.0, vm1 - _THRESH)
    fired = (v_exc > 0).astype(jnp.float32)
    vm2 = vm1 - vm1 * fired + 0.2 * fired
    acc1 = acc_ref[...] + vm1[:, -no:]
    spike_rate = jnp.mean(fired, axis=1, keepdims=True)
    input_norm = jnp.sqrt(jnp.sum(inp * inp, axis=1, keepdims=True)) * inject
    vexct_ref[...] = v_exc.T
    tail_ref[...] = jnp.concatenate(
        [vm2, acc1, inject, spike_rate, input_norm,
         jnp.zeros((b, tail_pad), jnp.float32)], axis=1)


def _make_sc_gather(b, e, n, ce):
    eper = e // _NW
    nchunks = eper // ce
    assert nchunks % 2 == 0
    kr = ce // _IR
    srows = n // _NS

    @functools.partial(
        pl.kernel, mesh=_mesh(),
        out_type=jax.ShapeDtypeStruct((e // _IR, _IR, b), jnp.float32),
        scratch_types=[
            pltpu.VMEM((2, kr, _IR, b), jnp.float32),
            pltpu.VMEM((2, kr, _IR), jnp.int32),
            pltpu.VMEM_SHARED((n, b), jnp.float32),
            pltpu.SemaphoreType.DMA,
            pltpu.SemaphoreType.DMA,
            pltpu.SemaphoreType.DMA,
            pltpu.SemaphoreType.DMA,
            pltpu.SemaphoreType.DMA,
            pltpu.SemaphoreType.DMA,
        ],
        compiler_params=_sc_params(),
    )
    def k(vexct_hbm, src3_hbm, gt_hbm, g_v, idx_v, shared,
          in0, in1, ga0, ga1, out0, out1):
        cid = lax.axis_index("c")
        sid = lax.axis_index("s")
        wid = sid * _NC + cid
        rb0 = (wid * eper) // _IR
        insems = (in0, in1)
        gasems = (ga0, ga1)
        outsems = (out0, out1)

        pltpu.sync_copy(vexct_hbm.at[pl.ds(sid * srows, srows)],
                        shared.at[pl.ds(sid * srows, srows)])
        plsc.subcore_barrier()

        def start_in(ci, p):
            pltpu.async_copy(src3_hbm.at[pl.ds(rb0 + ci * kr, kr)],
                             idx_v.at[p], insems[p])

        def wait_in(p):
            pltpu.make_async_copy(src3_hbm.at[pl.ds(0, kr)], idx_v.at[p],
                                  insems[p]).wait()

        def issue_gather(p):
            for j in range(kr):
                pltpu.async_copy(shared.at[idx_v.at[p, j]],
                                 g_v.at[p, j], gasems[p])

        def drain_gather(p):
            for j in range(kr):
                pltpu.make_async_copy(shared.at[idx_v.at[p, j]],
                                      g_v.at[p, j], gasems[p]).wait()

        def start_out(ci, p):
            pltpu.async_copy(g_v.at[p],
                             gt_hbm.at[pl.ds(rb0 + ci * kr, kr)],
                             outsems[p])

        def drain_out(p):
            pltpu.make_async_copy(g_v.at[p], gt_hbm.at[pl.ds(0, kr)],
                                  outsems[p]).wait()

        start_in(0, 0)

        def pair_body(i, _):
            start_in(2 * i + 1, 1)
            wait_in(0)

            @pl.when(i > 0)
            def _():
                drain_out(0)
            issue_gather(0)
            drain_gather(0)
            start_out(2 * i, 0)

            @pl.when(2 * i + 2 < nchunks)
            def _():
                start_in(2 * i + 2, 0)
            wait_in(1)

            @pl.when(i > 0)
            def _():
                drain_out(1)
            issue_gather(1)
            drain_gather(1)
            start_out(2 * i + 1, 1)
            return 0
        lax.fori_loop(0, nchunks // 2, pair_body, 0)
        drain_out(0)
        drain_out(1)

    return k


def _pass2_kernel(n_st_blocks, st_lo_ref, sv_lo_ref, l_lo_ref, st_hi_ref,
                  sv_hi_ref, l_hi_ref, gt_ref, tail_ref, out_ref,
                  stash_sthi, stash_svlo, stash_svhi):
    i = pl.program_id(0)
    last = 2 * n_st_blocks

    def half(st_ref, sv_ref, l_ref, g):
        st = st_ref[...]
        sv = sv_ref[...]
        lvals = l_ref[0]
        arrived = jnp.abs(st - lvals) <= (_ATOL + _RTOL * jnp.abs(lvals))
        stz = jnp.where(arrived, 0.0, st)
        svz = jnp.where(arrived, 0.0, sv)
        new = (g > 0) & (st == 0)
        st2 = stz + jnp.where(stz > 0, _DT * _VMAX, 0.0) \
                  + jnp.where(new, _DT * _VMAX, 0.0)
        sv2 = svz + jnp.where(new, g, 0.0)
        return st2, sv2

    @pl.when((i < last) & (i % 4 == 0))
    def _():
        b = st_lo_ref.shape[0]
        gt = gt_ref[...]
        st2_lo, sv2_lo = half(st_lo_ref, sv_lo_ref, l_lo_ref, gt[:, :b].T)
        st2_hi, sv2_hi = half(st_hi_ref, sv_hi_ref, l_hi_ref, gt[:, b:].T)
        out_ref[...] = st2_lo
        stash_sthi[...] = st2_hi
        stash_svlo[...] = sv2_lo
        stash_svhi[...] = sv2_hi

    @pl.when((i < last) & (i % 4 == 1))
    def _():
        out_ref[...] = stash_sthi[...]

    @pl.when((i < last) & (i % 4 == 2))
    def _():
        out_ref[...] = stash_svlo[...]

    @pl.when((i < last) & (i % 4 == 3))
    def _():
        out_ref[...] = stash_svhi[...]

    @pl.when(i == last)
    def _():
        out_ref[...] = tail_ref[...]


def _pack_idx(idx, e):
    return idx.reshape(2, e // _IR, _IR // 2).transpose(1, 2, 0).reshape(
        e // _IR, _IR)


def kernel(syn_travel, syn_value, vm, acc, input_current, L_e, W_e, phase,
           src, tgt):
    b, e = syn_travel.shape
    n = vm.shape[1]
    nh = input_current.shape[1]
    no = acc.shape[1]
    f32 = jnp.float32
    half = e // 2

    eba = _EB_A
    nblk_a = half // eba
    l3 = L_e.reshape(2 * nblk_a, 1, eba)
    w3 = W_e.reshape(2 * nblk_a, 1, eba)

    def lo2(i):
        return (0, i)

    def hi2(i):
        return (0, i + nblk_a)

    def lo3(i):
        return (i, 0, 0)

    def hi3(i):
        return (i + nblk_a, 0, 0)

    spikes_t = pl.pallas_call(
        _spikes_kernel,
        grid=(nblk_a,),
        in_specs=[
            pl.BlockSpec((b, eba), lo2),
            pl.BlockSpec((b, eba), lo2),
            pl.BlockSpec((1, 1, eba), lo3),
            pl.BlockSpec((1, 1, eba), lo3),
            pl.BlockSpec((b, eba), hi2),
            pl.BlockSpec((b, eba), hi2),
            pl.BlockSpec((1, 1, eba), hi3),
            pl.BlockSpec((1, 1, eba), hi3),
        ],
        out_specs=pl.BlockSpec((eba, _PB), lambda i: (i, 0)),
        out_shape=jax.ShapeDtypeStruct((half, _PB), f32),
    )(syn_travel, syn_value, l3, w3, syn_travel, syn_value, l3, w3)

    ce = 384
    isynt = _make_sc_scatter(b, e, n, ce)(
        spikes_t.reshape(e // _IR, _IR, b), _pack_idx(tgt, e))

    tail_cols = n + no + 3
    tail_pad = _EB_C - tail_cols
    vexct, tail = pl.pallas_call(
        functools.partial(_neuron_kernel, nh, no, tail_pad),
        out_shape=[
            jax.ShapeDtypeStruct((n, b), f32),
            jax.ShapeDtypeStruct((b, _EB_C), f32),
        ],
    )(isynt, vm, acc, input_current, phase.reshape(b, 1))

    gathered_t = _make_sc_gather(b, e, n, ce)(
        vexct, _pack_idx(src, e)).reshape(half, _PB)

    ebc = _EB_C
    nblk_c = e // ebc
    nhalf_c = nblk_c // 2
    out_cols = 2 * e + tail_cols
    l3c = L_e.reshape(nblk_c, 1, ebc)

    def q_of(i):
        return jnp.minimum(i // 4, nhalf_c - 1)

    def lo_map2(i):
        return (0, q_of(i))

    def hi_map2(i):
        return (0, nhalf_c + q_of(i))

    def lo_map3(i):
        return (q_of(i), 0, 0)

    def hi_map3(i):
        return (nhalf_c + q_of(i), 0, 0)

    def gt_map(i):
        return (q_of(i), 0)

    def out_map(i):
        return (0, jnp.where(i == 2 * nblk_c, 2 * nblk_c,
                             (i % 4) * nhalf_c + i // 4))

    out = pl.pallas_call(
        functools.partial(_pass2_kernel, nblk_c),
        grid=(2 * nblk_c + 1,),
        in_specs=[
            pl.BlockSpec((b, ebc), lo_map2),
            pl.BlockSpec((b, ebc), lo_map2),
            pl.BlockSpec((1, 1, ebc), lo_map3),
            pl.BlockSpec((b, ebc), hi_map2),
            pl.BlockSpec((b, ebc), hi_map2),
            pl.BlockSpec((1, 1, ebc), hi_map3),
            pl.BlockSpec((ebc, _PB), gt_map),
            pl.BlockSpec((b, _EB_C), lambda i: (0, 0)),
        ],
        out_specs=pl.BlockSpec((b, ebc), out_map),
        out_shape=jax.ShapeDtypeStruct((b, out_cols), f32),
        scratch_shapes=[pltpu.VMEM((b, ebc), f32),
                        pltpu.VMEM((b, ebc), f32),
                        pltpu.VMEM((b, ebc), f32)],
    )(syn_travel, syn_value, l3c, syn_travel, syn_value, l3c,
      gathered_t, tail)

    return out
